# Initial kernel scaffold; baseline (speedup 1.0000x reference)
import jax
import jax.numpy as jnp
from jax.experimental import pallas as pl


def kernel(X, w_bases_alpha, w_rel_alpha, w_bases_beta, w_rel_beta, bias_alpha, bias_beta, edge_index):
    raise NotImplementedError("write your pallas kernel here")


# speedup vs baseline: 5.9578x; 5.9578x over previous
import functools

import jax
import jax.numpy as jnp
from jax import lax
from jax.experimental import pallas as pl
from jax.experimental.pallas import tpu as pltpu
from jax.experimental.pallas import tpu_sc as plsc

N = 10000
E = 80000
R = 4
NB = 8
DIN = 128
DOUT = 128
DAUG = 144

NC = 2
NS = 16
CH = 128
EPAD = 81920
CPS = EPAD // (NS * CH)
RPC = R // NC
NPAD = 10112
ZROWS = NPAD // NS
WLAST = N - (NS - 1) * ZROWS

TN = 1000


def _sc_agg_body(xaug, rows, cols, zeros, out, idx_row, idx_col, gbuf, s_acc, sem):
    cid = lax.axis_index("c")
    sid = lax.axis_index("s")
    for rr in range(RPC):
        r = cid * RPC + rr
        pltpu.sync_copy(zeros, s_acc.at[pl.ds(sid * ZROWS, ZROWS)])
        plsc.subcore_barrier()

        def chunk(k, carry):
            base = (sid * CPS + k) * CH
            pltpu.sync_copy(rows.at[r, pl.ds(base, CH)], idx_row)
            pltpu.sync_copy(cols.at[r, pl.ds(base, CH)], idx_col)
            pltpu.async_copy(xaug.at[idx_col], gbuf, sem).wait()
            pltpu.sync_copy(gbuf, s_acc.at[idx_row], add=True)
            return carry

        lax.fori_loop(0, CPS, chunk, 0)
        plsc.subcore_barrier()

        @pl.when(sid < NS - 1)
        def _():
            pltpu.sync_copy(s_acc.at[pl.ds(sid * ZROWS, ZROWS)],
                            out.at[r, pl.ds(sid * ZROWS, ZROWS)])

        @pl.when(sid == NS - 1)
        def _():
            pltpu.sync_copy(s_acc.at[pl.ds((NS - 1) * ZROWS, WLAST)],
                            out.at[r, pl.ds((NS - 1) * ZROWS, WLAST)])

        plsc.subcore_barrier()


@functools.cache
def _sc_agg():
    return pl.kernel(
        _sc_agg_body,
        out_type=jax.ShapeDtypeStruct((R, N, DAUG), jnp.float32),
        mesh=plsc.VectorSubcoreMesh(core_axis_name="c", subcore_axis_name="s"),
        compiler_params=pltpu.CompilerParams(use_tc_tiling_on_sc=False),
        scratch_types=[
            pltpu.VMEM((CH,), jnp.int32),
            pltpu.VMEM((CH,), jnp.int32),
            pltpu.VMEM((CH, DAUG), jnp.float32),
            pltpu.VMEM_SHARED((NPAD, DAUG), jnp.float32),
            pltpu.SemaphoreType.DMA,
        ],
    )


def _softplus(x):
    m = jnp.maximum(x, 0.0)
    return m + jnp.log(jnp.exp(x - m) + jnp.exp(-m))


def _tc_body(s_ref, wra_ref, wrb_ref, wba_ref, wbb_ref, ba_ref, bb_ref,
             alpha_ref, beta_ref, wa_scr, wb_scr):
    @pl.when(pl.program_id(0) == 0)
    def _():
        for r in range(R):
            wa = jnp.zeros((DIN, DOUT), jnp.float32)
            wb = jnp.zeros((DIN, DOUT), jnp.float32)
            for b in range(NB):
                wa = wa + wra_ref[r, b] * wba_ref[b]
                wb = wb + wrb_ref[r, b] * wbb_ref[b]
            wa_scr[r] = wa
            wb_scr[r] = wb

    ya = jnp.zeros((TN, DOUT), jnp.float32)
    yb = jnp.zeros((TN, DOUT), jnp.float32)
    for r in range(R):
        s = s_ref[r]
        deg = jnp.sum(s[:, DIN:DAUG], axis=1, keepdims=True)
        t = s[:, :DIN] * (1.0 / (deg + 1e-8))
        ya = ya + jnp.dot(t, wa_scr[r], preferred_element_type=jnp.float32)
        yb = yb + jnp.dot(t, wb_scr[r], preferred_element_type=jnp.float32)
    xa = jnp.maximum(ya, 0.0) + ba_ref[...]
    xb = jnp.maximum(yb, 0.0) + bb_ref[...]
    alpha_ref[...] = 1.01 + _softplus(xa)
    beta_ref[...] = 1.01 + _softplus(xb)


_tc_combine = pl.pallas_call(
    _tc_body,
    grid=(N // TN,),
    in_specs=[
        pl.BlockSpec((R, TN, DAUG), lambda i: (0, i, 0)),
        pl.BlockSpec(memory_space=pltpu.SMEM),
        pl.BlockSpec(memory_space=pltpu.SMEM),
        pl.BlockSpec((NB, DIN, DOUT), lambda i: (0, 0, 0)),
        pl.BlockSpec((NB, DIN, DOUT), lambda i: (0, 0, 0)),
        pl.BlockSpec((1, DOUT), lambda i: (0, 0)),
        pl.BlockSpec((1, DOUT), lambda i: (0, 0)),
    ],
    out_specs=[
        pl.BlockSpec((TN, DOUT), lambda i: (i, 0)),
        pl.BlockSpec((TN, DOUT), lambda i: (i, 0)),
    ],
    out_shape=[
        jax.ShapeDtypeStruct((N, DOUT), jnp.float32),
        jax.ShapeDtypeStruct((N, DOUT), jnp.float32),
    ],
    scratch_shapes=[
        pltpu.VMEM((R, DIN, DOUT), jnp.float32),
        pltpu.VMEM((R, DIN, DOUT), jnp.float32),
    ],
)


def kernel(X, w_bases_alpha, w_rel_alpha, w_bases_beta, w_rel_beta,
           bias_alpha, bias_beta, edge_index):
    Xs = jnp.nan_to_num(X, nan=0.0)
    xaug = jnp.concatenate(
        [Xs, jnp.ones((N, 1), jnp.float32), jnp.zeros((N, DAUG - DIN - 1), jnp.float32)],
        axis=1)
    pad = EPAD - E
    rows = jnp.concatenate(
        [edge_index[:, 0, :], jnp.full((R, pad), N, jnp.int32)], axis=1)
    cols = jnp.concatenate(
        [edge_index[:, 1, :], jnp.zeros((R, pad), jnp.int32)], axis=1)
    zeros = jnp.zeros((ZROWS, DAUG), jnp.float32)

    s = _sc_agg()(xaug, rows, cols, zeros)
    alpha, beta = _tc_combine(
        s, w_rel_alpha, w_rel_beta, w_bases_alpha, w_bases_beta,
        bias_alpha.reshape(1, DOUT), bias_beta.reshape(1, DOUT))
    return (alpha, beta)
